# baseline (speedup 1.0000x reference)
import jax
import jax.numpy as jnp
from jax import lax
from jax.experimental import pallas as pl
from jax.experimental.pallas import tpu as pltpu
from jax.experimental.pallas import tpu_sc as plsc

D = 64
DP = 128
SCALE = float(D) ** 0.5
NC, NS = 2, 16
NW = NC * NS
B_TOTAL = 4096 * 200
PER_W = B_TOTAL // NW
C = 64
NBUF = 5
NCHUNK = PER_W // C
NGROUP = NCHUNK // NBUF


def _embed_body(x_hbm, tab_hbm, out_hbm, idx_v, rowsP, rows64,
                sem_i, sem_g, sem_o):
    wid = lax.axis_index("s") * NC + lax.axis_index("c")
    base = wid * PER_W

    def fire_idx(c, b):
        pltpu.async_copy(
            x_hbm.at[pl.ds(base + c * C, C)], idx_v.at[b], sem_i.at[b]
        )

    def wait_idx(b):
        pltpu.make_async_copy(
            x_hbm.at[pl.ds(0, C)], idx_v.at[b], sem_i.at[b]
        ).wait()

    def fire_gather(b):
        pltpu.async_copy(
            tab_hbm.at[idx_v.at[b]], rowsP.at[b], sem_g.at[b]
        )

    def wait_gather(b):
        pltpu.make_async_copy(
            tab_hbm.at[pl.ds(0, C)], rowsP.at[b], sem_g.at[b]
        ).wait()

    def fire_out(c, b):
        pltpu.async_copy(
            rows64.at[b], out_hbm.at[pl.ds(base + c * C, C)], sem_o.at[b]
        )

    def wait_out(b):
        pltpu.make_async_copy(
            rows64.at[b], out_hbm.at[pl.ds(0, C)], sem_o.at[b]
        ).wait()

    for b in range(NBUF):
        fire_idx(b, b)
    for b in range(NBUF):
        wait_idx(b)
        fire_gather(b)

    def group(g, carry):
        for b in range(NBUF):
            c = g * NBUF + b
            wait_gather(b)

            @pl.when(g + 1 < NGROUP)
            def _():
                fire_idx(c + NBUF, b)

            @pl.when(g > 0)
            def _():
                wait_out(b)

            @plsc.parallel_loop(0, C, step=1, unroll=8)
            def _scale(i):
                for j in range(D // 16):
                    sl = pl.ds(j * 16, 16)
                    rows64[b, i, sl] = rowsP[b, i, sl] * SCALE

            fire_out(c, b)

            @pl.when(g + 1 < NGROUP)
            def _():
                wait_idx(b)
                fire_gather(b)

        return carry

    lax.fori_loop(0, NGROUP, group, 0)

    for b in range(NBUF):
        wait_out(b)


@jax.jit
def kernel(x, table):
    xf = x.reshape(-1).astype(jnp.int32)
    tabP = jnp.pad(table, ((0, 0), (0, DP - D)))
    mesh = plsc.VectorSubcoreMesh(
        core_axis_name="c", subcore_axis_name="s",
        num_cores=NC, num_subcores=NS,
    )
    out = pl.kernel(
        _embed_body,
        out_type=jax.ShapeDtypeStruct((B_TOTAL, D), jnp.float32),
        mesh=mesh,
        scratch_types=[
            pltpu.VMEM((NBUF, C), jnp.int32),
            pltpu.VMEM((NBUF, C, DP), jnp.float32),
            pltpu.VMEM((NBUF, C, D), jnp.float32),
            pltpu.SemaphoreType.DMA((NBUF,)),
            pltpu.SemaphoreType.DMA((NBUF,)),
            pltpu.SemaphoreType.DMA((NBUF,)),
        ],
        compiler_params=pltpu.CompilerParams(use_tc_tiling_on_sc=True),
    )(xf, tabP)
    return out.reshape(x.shape[0], x.shape[1], D)


# speedup vs baseline: 1.0385x; 1.0385x over previous
import jax
import jax.numpy as jnp
from jax import lax
from jax.experimental import pallas as pl
from jax.experimental.pallas import tpu as pltpu
from jax.experimental.pallas import tpu_sc as plsc

D = 64
DP = 128
SCALE = float(D) ** 0.5
VOCAB = 1000000
TBLK = 2048
NC, NS = 2, 16
NW = NC * NS
B_TOTAL = 4096 * 200
PER_W = B_TOTAL // NW
C = 64
NBUF = 5
NCHUNK = PER_W // C
NGROUP = NCHUNK // NBUF


def _prep_body(tabT_ref, out_ref):
    r = lax.broadcasted_iota(jnp.int32, (D, D), 0)
    c = lax.broadcasted_iota(jnp.int32, (D, D), 1)
    eye_scaled = jnp.where(r == c, SCALE, 0.0).astype(jnp.float32)
    a = tabT_ref[...]
    out_ref[:, :D] = lax.dot_general(
        a, eye_scaled, (((0,), (0,)), ((), ())),
        preferred_element_type=jnp.float32,
    )


def _embed_body(x_hbm, tab_hbm, out_hbm, idx_v, rowsP, rows64,
                sem_i, sem_g, sem_o):
    wid = lax.axis_index("s") * NC + lax.axis_index("c")
    base = wid * PER_W

    def fire_idx(c_, b):
        pltpu.async_copy(
            x_hbm.at[pl.ds(base + c_ * C, C)], idx_v.at[b], sem_i.at[b]
        )

    def wait_idx(b):
        pltpu.make_async_copy(
            x_hbm.at[pl.ds(0, C)], idx_v.at[b], sem_i.at[b]
        ).wait()

    def fire_gather(b):
        pltpu.async_copy(
            tab_hbm.at[idx_v.at[b]], rowsP.at[b], sem_g.at[b]
        )

    def wait_gather(b):
        pltpu.make_async_copy(
            tab_hbm.at[pl.ds(0, C)], rowsP.at[b], sem_g.at[b]
        ).wait()

    def fire_out(c_, b):
        pltpu.async_copy(
            rows64.at[b], out_hbm.at[pl.ds(base + c_ * C, C)], sem_o.at[b]
        )

    def wait_out(b):
        pltpu.make_async_copy(
            rows64.at[b], out_hbm.at[pl.ds(0, C)], sem_o.at[b]
        ).wait()

    for b in range(NBUF):
        fire_idx(b, b)
    for b in range(NBUF):
        wait_idx(b)
        fire_gather(b)

    def group(g, carry):
        for b in range(NBUF):
            c_ = g * NBUF + b
            wait_gather(b)

            @pl.when(g + 1 < NGROUP)
            def _():
                fire_idx(c_ + NBUF, b)

            @pl.when(g > 0)
            def _():
                wait_out(b)

            @plsc.parallel_loop(0, C, step=1, unroll=8)
            def _compact(i):
                for j in range(D // 16):
                    sl = pl.ds(j * 16, 16)
                    rows64[b, i, sl] = rowsP[b, i, sl]

            fire_out(c_, b)

            @pl.when(g + 1 < NGROUP)
            def _():
                wait_idx(b)
                fire_gather(b)

        return carry

    lax.fori_loop(0, NGROUP, group, 0)

    for b in range(NBUF):
        wait_out(b)


@jax.jit
def kernel(x, table):
    xf = x.reshape(-1).astype(jnp.int32)

    tabP = pl.pallas_call(
        _prep_body,
        grid=(pl.cdiv(VOCAB, TBLK),),
        in_specs=[pl.BlockSpec((D, TBLK), lambda i: (0, i))],
        out_specs=pl.BlockSpec((TBLK, DP), lambda i: (i, 0)),
        out_shape=jax.ShapeDtypeStruct((VOCAB, DP), jnp.float32),
    )(table.T)

    mesh = plsc.VectorSubcoreMesh(
        core_axis_name="c", subcore_axis_name="s",
        num_cores=NC, num_subcores=NS,
    )
    out = pl.kernel(
        _embed_body,
        out_type=jax.ShapeDtypeStruct((B_TOTAL, D), jnp.float32),
        mesh=mesh,
        scratch_types=[
            pltpu.VMEM((NBUF, C), jnp.int32),
            pltpu.VMEM((NBUF, C, DP), jnp.float32),
            pltpu.VMEM((NBUF, C, D), jnp.float32),
            pltpu.SemaphoreType.DMA((NBUF,)),
            pltpu.SemaphoreType.DMA((NBUF,)),
            pltpu.SemaphoreType.DMA((NBUF,)),
        ],
        compiler_params=pltpu.CompilerParams(use_tc_tiling_on_sc=True),
    )(xf, tabP)
    return out.reshape(x.shape[0], x.shape[1], D)


# speedup vs baseline: 1.0392x; 1.0007x over previous
import jax
import jax.numpy as jnp
from jax import lax
from jax.experimental import pallas as pl
from jax.experimental.pallas import tpu as pltpu
from jax.experimental.pallas import tpu_sc as plsc

D = 64
DP = 128
SCALE = float(D) ** 0.5
VOCAB = 1000000
TBLK = 2048
NC, NS = 2, 16
NW = NC * NS
B_TOTAL = 4096 * 200
PER_W = B_TOTAL // NW
C = 64
NBUF = 5
NCHUNK = PER_W // C
NGROUP = NCHUNK // NBUF


def _prep_body(tabT_ref, out_ref):
    r = lax.broadcasted_iota(jnp.int32, (D, DP), 0)
    c = lax.broadcasted_iota(jnp.int32, (D, DP), 1)
    eye_scaled = jnp.where(r == c, SCALE, 0.0).astype(jnp.float32)
    a = tabT_ref[...]
    out_ref[...] = lax.dot_general(
        a, eye_scaled, (((0,), (0,)), ((), ())),
        preferred_element_type=jnp.float32,
    )


def _embed_body(x_hbm, tab_hbm, out_hbm, idx_v, rowsP, rows64,
                sem_i, sem_g, sem_o):
    wid = lax.axis_index("s") * NC + lax.axis_index("c")
    base = wid * PER_W

    def fire_idx(c_, b):
        pltpu.async_copy(
            x_hbm.at[pl.ds(base + c_ * C, C)], idx_v.at[b], sem_i.at[b]
        )

    def wait_idx(b):
        pltpu.make_async_copy(
            x_hbm.at[pl.ds(0, C)], idx_v.at[b], sem_i.at[b]
        ).wait()

    def fire_gather(b):
        pltpu.async_copy(
            tab_hbm.at[idx_v.at[b]], rowsP.at[b], sem_g.at[b]
        )

    def wait_gather(b):
        pltpu.make_async_copy(
            tab_hbm.at[pl.ds(0, C)], rowsP.at[b], sem_g.at[b]
        ).wait()

    def fire_out(c_, b):
        pltpu.async_copy(
            rows64.at[b], out_hbm.at[pl.ds(base + c_ * C, C)], sem_o.at[b]
        )

    def wait_out(b):
        pltpu.make_async_copy(
            rows64.at[b], out_hbm.at[pl.ds(0, C)], sem_o.at[b]
        ).wait()

    for b in range(NBUF):
        fire_idx(b, b)
    for b in range(NBUF):
        wait_idx(b)
        fire_gather(b)

    def group(g, carry):
        for b in range(NBUF):
            c_ = g * NBUF + b
            wait_gather(b)

            @pl.when(g + 1 < NGROUP)
            def _():
                fire_idx(c_ + NBUF, b)

            @pl.when(g > 0)
            def _():
                wait_out(b)

            @plsc.parallel_loop(0, C, step=1, unroll=8)
            def _compact(i):
                for j in range(D // 16):
                    sl = pl.ds(j * 16, 16)
                    rows64[b, i, sl] = rowsP[b, i, sl]

            fire_out(c_, b)

            @pl.when(g + 1 < NGROUP)
            def _():
                wait_idx(b)
                fire_gather(b)

        return carry

    lax.fori_loop(0, NGROUP, group, 0)

    for b in range(NBUF):
        wait_out(b)


@jax.jit
def kernel(x, table):
    xf = x.reshape(-1).astype(jnp.int32)

    tabP = pl.pallas_call(
        _prep_body,
        grid=(pl.cdiv(VOCAB, TBLK),),
        in_specs=[pl.BlockSpec((D, TBLK), lambda i: (0, i))],
        out_specs=pl.BlockSpec((TBLK, DP), lambda i: (i, 0)),
        out_shape=jax.ShapeDtypeStruct((VOCAB, DP), jnp.float32),
        compiler_params=pltpu.CompilerParams(
            fuse_transposed_lhs_in_matmul=True
        ),
    )(table.T)

    mesh = plsc.VectorSubcoreMesh(
        core_axis_name="c", subcore_axis_name="s",
        num_cores=NC, num_subcores=NS,
    )
    out = pl.kernel(
        _embed_body,
        out_type=jax.ShapeDtypeStruct((B_TOTAL, D), jnp.float32),
        mesh=mesh,
        scratch_types=[
            pltpu.VMEM((NBUF, C), jnp.int32),
            pltpu.VMEM((NBUF, C, DP), jnp.float32),
            pltpu.VMEM((NBUF, C, D), jnp.float32),
            pltpu.SemaphoreType.DMA((NBUF,)),
            pltpu.SemaphoreType.DMA((NBUF,)),
            pltpu.SemaphoreType.DMA((NBUF,)),
        ],
        compiler_params=pltpu.CompilerParams(use_tc_tiling_on_sc=True),
    )(xf, tabP)
    return out.reshape(x.shape[0], x.shape[1], D)


# speedup vs baseline: 1.3302x; 1.2800x over previous
import jax
import jax.numpy as jnp
from jax import lax
from jax.experimental import pallas as pl
from jax.experimental.pallas import tpu as pltpu
from jax.experimental.pallas import tpu_sc as plsc

D = 64
DP = 128
SCALE = float(D) ** 0.5
VOCAB = 1000000
TBLK = 8192
NC, NS = 2, 16
NW = NC * NS
B_TOTAL = 4096 * 200
PER_W = B_TOTAL // NW
C = 64
NBUF = 5
NCHUNK = PER_W // C
NGROUP = NCHUNK // NBUF


def _prep_body(tabT_ref, out_ref):
    r = lax.broadcasted_iota(jnp.int32, (D, DP), 0)
    c = lax.broadcasted_iota(jnp.int32, (D, DP), 1)
    eye_scaled = jnp.where(r == c, SCALE, 0.0).astype(jnp.float32)
    a = tabT_ref[...]
    out_ref[...] = lax.dot_general(
        a, eye_scaled, (((0,), (0,)), ((), ())),
        preferred_element_type=jnp.float32,
    )


def _embed_body(x_hbm, tab_hbm, out_hbm, idx_v, rowsP, rows64,
                sem_i, sem_g, sem_o):
    wid = lax.axis_index("s") * NC + lax.axis_index("c")
    base = wid * PER_W

    def fire_idx(c_, b):
        pltpu.async_copy(
            x_hbm.at[pl.ds(base + c_ * C, C)], idx_v.at[b], sem_i.at[b]
        )

    def wait_idx(b):
        pltpu.make_async_copy(
            x_hbm.at[pl.ds(0, C)], idx_v.at[b], sem_i.at[b]
        ).wait()

    def fire_gather(b):
        pltpu.async_copy(
            tab_hbm.at[idx_v.at[b]], rowsP.at[b], sem_g.at[b]
        )

    def wait_gather(b):
        pltpu.make_async_copy(
            tab_hbm.at[pl.ds(0, C)], rowsP.at[b], sem_g.at[b]
        ).wait()

    def fire_out(c_, b):
        pltpu.async_copy(
            rows64.at[b], out_hbm.at[pl.ds(base + c_ * C, C)], sem_o.at[b]
        )

    def wait_out(b):
        pltpu.make_async_copy(
            rows64.at[b], out_hbm.at[pl.ds(0, C)], sem_o.at[b]
        ).wait()

    for b in range(NBUF):
        fire_idx(b, b)
    for b in range(NBUF):
        wait_idx(b)
        fire_gather(b)

    def group(g, carry):
        for b in range(NBUF):
            c_ = g * NBUF + b
            wait_gather(b)

            @pl.when(g + 1 < NGROUP)
            def _():
                fire_idx(c_ + NBUF, b)

            @pl.when(g > 0)
            def _():
                wait_out(b)

            @plsc.parallel_loop(0, C, step=1, unroll=8)
            def _compact(i):
                for j in range(D // 16):
                    sl = pl.ds(j * 16, 16)
                    rows64[b, i, sl] = rowsP[b, i, sl]

            fire_out(c_, b)

            @pl.when(g + 1 < NGROUP)
            def _():
                wait_idx(b)
                fire_gather(b)

        return carry

    lax.fori_loop(0, NGROUP, group, 0)

    for b in range(NBUF):
        wait_out(b)


@jax.jit
def kernel(x, table):
    xf = x.reshape(-1).astype(jnp.int32)

    tabP = pl.pallas_call(
        _prep_body,
        grid=(pl.cdiv(VOCAB, TBLK),),
        in_specs=[pl.BlockSpec((D, TBLK), lambda i: (0, i))],
        out_specs=pl.BlockSpec((TBLK, DP), lambda i: (i, 0)),
        out_shape=jax.ShapeDtypeStruct((VOCAB, DP), jnp.float32),
        compiler_params=pltpu.CompilerParams(
            fuse_transposed_lhs_in_matmul=True
        ),
    )(table.T)

    mesh = plsc.VectorSubcoreMesh(
        core_axis_name="c", subcore_axis_name="s",
        num_cores=NC, num_subcores=NS,
    )
    out = pl.kernel(
        _embed_body,
        out_type=jax.ShapeDtypeStruct((B_TOTAL, D), jnp.float32),
        mesh=mesh,
        scratch_types=[
            pltpu.VMEM((NBUF, C), jnp.int32),
            pltpu.VMEM((NBUF, C, DP), jnp.float32),
            pltpu.VMEM((NBUF, C, D), jnp.float32),
            pltpu.SemaphoreType.DMA((NBUF,)),
            pltpu.SemaphoreType.DMA((NBUF,)),
            pltpu.SemaphoreType.DMA((NBUF,)),
        ],
        compiler_params=pltpu.CompilerParams(use_tc_tiling_on_sc=True),
    )(xf, tabP)
    return out.reshape(x.shape[0], x.shape[1], D)


# speedup vs baseline: 1.3768x; 1.0351x over previous
import jax
import jax.numpy as jnp
from jax import lax
from jax.experimental import pallas as pl
from jax.experimental.pallas import tpu as pltpu
from jax.experimental.pallas import tpu_sc as plsc

D = 64
DP = 128
SCALE = float(D) ** 0.5
VOCAB = 1000000
TBLK = 16384
NC, NS = 2, 16
NW = NC * NS
B_TOTAL = 4096 * 200
PER_W = B_TOTAL // NW
C = 64
NBUF = 5
NCHUNK = PER_W // C
NGROUP = NCHUNK // NBUF


def _prep_body(tabT_ref, out_ref):
    r = lax.broadcasted_iota(jnp.int32, (D, DP), 0)
    c = lax.broadcasted_iota(jnp.int32, (D, DP), 1)
    eye_scaled = jnp.where(r == c, SCALE, 0.0).astype(jnp.float32)
    a = tabT_ref[...]
    out_ref[...] = lax.dot_general(
        a, eye_scaled, (((0,), (0,)), ((), ())),
        preferred_element_type=jnp.float32,
    )


def _embed_body(x_hbm, tab_hbm, out_hbm, idx_v, rowsP, rows64,
                sem_i, sem_g, sem_o):
    wid = lax.axis_index("s") * NC + lax.axis_index("c")
    base = wid * PER_W

    def fire_idx(c_, b):
        pltpu.async_copy(
            x_hbm.at[pl.ds(base + c_ * C, C)], idx_v.at[b], sem_i.at[b]
        )

    def wait_idx(b):
        pltpu.make_async_copy(
            x_hbm.at[pl.ds(0, C)], idx_v.at[b], sem_i.at[b]
        ).wait()

    def fire_gather(b):
        pltpu.async_copy(
            tab_hbm.at[idx_v.at[b]], rowsP.at[b], sem_g.at[b]
        )

    def wait_gather(b):
        pltpu.make_async_copy(
            tab_hbm.at[pl.ds(0, C)], rowsP.at[b], sem_g.at[b]
        ).wait()

    def fire_out(c_, b):
        pltpu.async_copy(
            rows64.at[b], out_hbm.at[pl.ds(base + c_ * C, C)], sem_o.at[b]
        )

    def wait_out(b):
        pltpu.make_async_copy(
            rows64.at[b], out_hbm.at[pl.ds(0, C)], sem_o.at[b]
        ).wait()

    for b in range(NBUF):
        fire_idx(b, b)
    for b in range(NBUF):
        wait_idx(b)
        fire_gather(b)

    def group(g, carry):
        for b in range(NBUF):
            c_ = g * NBUF + b
            wait_gather(b)

            @pl.when(g + 1 < NGROUP)
            def _():
                fire_idx(c_ + NBUF, b)

            @pl.when(g > 0)
            def _():
                wait_out(b)

            @plsc.parallel_loop(0, C, step=1, unroll=8)
            def _compact(i):
                for j in range(D // 16):
                    sl = pl.ds(j * 16, 16)
                    rows64[b, i, sl] = rowsP[b, i, sl]

            fire_out(c_, b)

            @pl.when(g + 1 < NGROUP)
            def _():
                wait_idx(b)
                fire_gather(b)

        return carry

    lax.fori_loop(0, NGROUP, group, 0)

    for b in range(NBUF):
        wait_out(b)


@jax.jit
def kernel(x, table):
    xf = x.reshape(-1).astype(jnp.int32)

    tabP = pl.pallas_call(
        _prep_body,
        grid=(pl.cdiv(VOCAB, TBLK),),
        in_specs=[pl.BlockSpec((D, TBLK), lambda i: (0, i))],
        out_specs=pl.BlockSpec((TBLK, DP), lambda i: (i, 0)),
        out_shape=jax.ShapeDtypeStruct((VOCAB, DP), jnp.float32),
        compiler_params=pltpu.CompilerParams(
            fuse_transposed_lhs_in_matmul=True
        ),
    )(table.T)

    mesh = plsc.VectorSubcoreMesh(
        core_axis_name="c", subcore_axis_name="s",
        num_cores=NC, num_subcores=NS,
    )
    out = pl.kernel(
        _embed_body,
        out_type=jax.ShapeDtypeStruct((B_TOTAL, D), jnp.float32),
        mesh=mesh,
        scratch_types=[
            pltpu.VMEM((NBUF, C), jnp.int32),
            pltpu.VMEM((NBUF, C, DP), jnp.float32),
            pltpu.VMEM((NBUF, C, D), jnp.float32),
            pltpu.SemaphoreType.DMA((NBUF,)),
            pltpu.SemaphoreType.DMA((NBUF,)),
            pltpu.SemaphoreType.DMA((NBUF,)),
        ],
        compiler_params=pltpu.CompilerParams(use_tc_tiling_on_sc=True),
    )(xf, tabP)
    return out.reshape(x.shape[0], x.shape[1], D)


# speedup vs baseline: 1.3866x; 1.0071x over previous
import jax
import jax.numpy as jnp
from jax import lax
from jax.experimental import pallas as pl
from jax.experimental.pallas import tpu as pltpu
from jax.experimental.pallas import tpu_sc as plsc

D = 64
DP = 128
SCALE = float(D) ** 0.5
VOCAB = 1000000
TBLK = 32768
NC, NS = 2, 16
NW = NC * NS
B_TOTAL = 4096 * 200
PER_W = B_TOTAL // NW
C = 64
NBUF = 5
NCHUNK = PER_W // C
NGROUP = NCHUNK // NBUF


def _prep_body(tabT_ref, out_ref):
    r = lax.broadcasted_iota(jnp.int32, (D, DP), 0)
    c = lax.broadcasted_iota(jnp.int32, (D, DP), 1)
    eye_scaled = jnp.where(r == c, SCALE, 0.0).astype(jnp.float32)
    a = tabT_ref[...]
    out_ref[...] = lax.dot_general(
        a, eye_scaled, (((0,), (0,)), ((), ())),
        preferred_element_type=jnp.float32,
    )


def _embed_body(x_hbm, tab_hbm, out_hbm, idx_v, rowsP, rows64,
                sem_i, sem_g, sem_o):
    wid = lax.axis_index("s") * NC + lax.axis_index("c")
    base = wid * PER_W

    def fire_idx(c_, b):
        pltpu.async_copy(
            x_hbm.at[pl.ds(base + c_ * C, C)], idx_v.at[b], sem_i.at[b]
        )

    def wait_idx(b):
        pltpu.make_async_copy(
            x_hbm.at[pl.ds(0, C)], idx_v.at[b], sem_i.at[b]
        ).wait()

    def fire_gather(b):
        pltpu.async_copy(
            tab_hbm.at[idx_v.at[b]], rowsP.at[b], sem_g.at[b]
        )

    def wait_gather(b):
        pltpu.make_async_copy(
            tab_hbm.at[pl.ds(0, C)], rowsP.at[b], sem_g.at[b]
        ).wait()

    def fire_out(c_, b):
        pltpu.async_copy(
            rows64.at[b], out_hbm.at[pl.ds(base + c_ * C, C)], sem_o.at[b]
        )

    def wait_out(b):
        pltpu.make_async_copy(
            rows64.at[b], out_hbm.at[pl.ds(0, C)], sem_o.at[b]
        ).wait()

    for b in range(NBUF):
        fire_idx(b, b)
    for b in range(NBUF):
        wait_idx(b)
        fire_gather(b)

    def group(g, carry):
        for b in range(NBUF):
            c_ = g * NBUF + b
            wait_gather(b)

            @pl.when(g + 1 < NGROUP)
            def _():
                fire_idx(c_ + NBUF, b)

            @pl.when(g > 0)
            def _():
                wait_out(b)

            @plsc.parallel_loop(0, C, step=1, unroll=8)
            def _compact(i):
                for j in range(D // 16):
                    sl = pl.ds(j * 16, 16)
                    rows64[b, i, sl] = rowsP[b, i, sl]

            fire_out(c_, b)

            @pl.when(g + 1 < NGROUP)
            def _():
                wait_idx(b)
                fire_gather(b)

        return carry

    lax.fori_loop(0, NGROUP, group, 0)

    for b in range(NBUF):
        wait_out(b)


@jax.jit
def kernel(x, table):
    xf = x.reshape(-1).astype(jnp.int32)

    tabP = pl.pallas_call(
        _prep_body,
        grid=(pl.cdiv(VOCAB, TBLK),),
        in_specs=[pl.BlockSpec((D, TBLK), lambda i: (0, i))],
        out_specs=pl.BlockSpec((TBLK, DP), lambda i: (i, 0)),
        out_shape=jax.ShapeDtypeStruct((VOCAB, DP), jnp.float32),
        compiler_params=pltpu.CompilerParams(
            fuse_transposed_lhs_in_matmul=True
        ),
    )(table.T)

    mesh = plsc.VectorSubcoreMesh(
        core_axis_name="c", subcore_axis_name="s",
        num_cores=NC, num_subcores=NS,
    )
    out = pl.kernel(
        _embed_body,
        out_type=jax.ShapeDtypeStruct((B_TOTAL, D), jnp.float32),
        mesh=mesh,
        scratch_types=[
            pltpu.VMEM((NBUF, C), jnp.int32),
            pltpu.VMEM((NBUF, C, DP), jnp.float32),
            pltpu.VMEM((NBUF, C, D), jnp.float32),
            pltpu.SemaphoreType.DMA((NBUF,)),
            pltpu.SemaphoreType.DMA((NBUF,)),
            pltpu.SemaphoreType.DMA((NBUF,)),
        ],
        compiler_params=pltpu.CompilerParams(use_tc_tiling_on_sc=True),
    )(xf, tabP)
    return out.reshape(x.shape[0], x.shape[1], D)


# speedup vs baseline: 1.3908x; 1.0031x over previous
import jax
import jax.numpy as jnp
from jax import lax
from jax.experimental import pallas as pl
from jax.experimental.pallas import tpu as pltpu
from jax.experimental.pallas import tpu_sc as plsc

D = 64
DP = 128
SCALE = float(D) ** 0.5
VOCAB = 1000000
TBLK = 32768
NC, NS = 2, 16
NW = NC * NS
B_TOTAL = 4096 * 200
PER_W = B_TOTAL // NW
C = 80
NBUF = 4
NCHUNK = PER_W // C
NGROUP = NCHUNK // NBUF


def _prep_body(tabT_ref, out_ref):
    r = lax.broadcasted_iota(jnp.int32, (D, DP), 0)
    c = lax.broadcasted_iota(jnp.int32, (D, DP), 1)
    eye_scaled = jnp.where(r == c, SCALE, 0.0).astype(jnp.float32)
    a = tabT_ref[...]
    out_ref[...] = lax.dot_general(
        a, eye_scaled, (((0,), (0,)), ((), ())),
        preferred_element_type=jnp.float32,
    )


def _embed_body(x_hbm, tab_hbm, out_hbm, idx_v, rowsP, rows64,
                sem_i, sem_g, sem_o):
    wid = lax.axis_index("s") * NC + lax.axis_index("c")
    base = wid * PER_W

    def fire_idx(c_, b):
        pltpu.async_copy(
            x_hbm.at[pl.ds(base + c_ * C, C)], idx_v.at[b], sem_i.at[b]
        )

    def wait_idx(b):
        pltpu.make_async_copy(
            x_hbm.at[pl.ds(0, C)], idx_v.at[b], sem_i.at[b]
        ).wait()

    def fire_gather(b):
        pltpu.async_copy(
            tab_hbm.at[idx_v.at[b]], rowsP.at[b], sem_g.at[b]
        )

    def wait_gather(b):
        pltpu.make_async_copy(
            tab_hbm.at[pl.ds(0, C)], rowsP.at[b], sem_g.at[b]
        ).wait()

    def fire_out(c_, b):
        pltpu.async_copy(
            rows64.at[b], out_hbm.at[pl.ds(base + c_ * C, C)], sem_o.at[b]
        )

    def wait_out(b):
        pltpu.make_async_copy(
            rows64.at[b], out_hbm.at[pl.ds(0, C)], sem_o.at[b]
        ).wait()

    for b in range(NBUF):
        fire_idx(b, b)
    for b in range(NBUF):
        wait_idx(b)
        fire_gather(b)

    def group(g, carry):
        for b in range(NBUF):
            c_ = g * NBUF + b
            wait_gather(b)

            @pl.when(g + 1 < NGROUP)
            def _():
                fire_idx(c_ + NBUF, b)

            @pl.when(g > 0)
            def _():
                wait_out(b)

            @plsc.parallel_loop(0, C, step=1, unroll=8)
            def _compact(i):
                for j in range(D // 16):
                    sl = pl.ds(j * 16, 16)
                    rows64[b, i, sl] = rowsP[b, i, sl]

            fire_out(c_, b)

            @pl.when(g + 1 < NGROUP)
            def _():
                wait_idx(b)
                fire_gather(b)

        return carry

    lax.fori_loop(0, NGROUP, group, 0)

    for b in range(NBUF):
        wait_out(b)


@jax.jit
def kernel(x, table):
    xf = x.reshape(-1).astype(jnp.int32)

    tabP = pl.pallas_call(
        _prep_body,
        grid=(pl.cdiv(VOCAB, TBLK),),
        in_specs=[pl.BlockSpec((D, TBLK), lambda i: (0, i))],
        out_specs=pl.BlockSpec((TBLK, DP), lambda i: (i, 0)),
        out_shape=jax.ShapeDtypeStruct((VOCAB, DP), jnp.float32),
        compiler_params=pltpu.CompilerParams(
            fuse_transposed_lhs_in_matmul=True
        ),
    )(table.T)

    mesh = plsc.VectorSubcoreMesh(
        core_axis_name="c", subcore_axis_name="s",
        num_cores=NC, num_subcores=NS,
    )
    out = pl.kernel(
        _embed_body,
        out_type=jax.ShapeDtypeStruct((B_TOTAL, D), jnp.float32),
        mesh=mesh,
        scratch_types=[
            pltpu.VMEM((NBUF, C), jnp.int32),
            pltpu.VMEM((NBUF, C, DP), jnp.float32),
            pltpu.VMEM((NBUF, C, D), jnp.float32),
            pltpu.SemaphoreType.DMA((NBUF,)),
            pltpu.SemaphoreType.DMA((NBUF,)),
            pltpu.SemaphoreType.DMA((NBUF,)),
        ],
        compiler_params=pltpu.CompilerParams(use_tc_tiling_on_sc=True),
    )(xf, tabP)
    return out.reshape(x.shape[0], x.shape[1], D)
